# baseline (speedup 1.0000x reference)
import functools

import jax
import jax.numpy as jnp
from jax.experimental import pallas as pl


def _dcn_loss_kernel(ea_ref, cta_ref, out_ref, *, n_total):
    i = pl.program_id(0)

    ea = ea_ref[...]
    cta = cta_ref[...]

    paug = jax.lax.dot_general(
        ea, cta, (((1,), (0,)), ((), ())),
        preferred_element_type=jnp.float32,
    )

    rowmax = jnp.max(paug, axis=1, keepdims=True)
    partial = jnp.sum(rowmax) * (-2.0 / n_total)

    @pl.when(i == 0)
    def _():
        out_ref[...] = jnp.zeros((1, 1), jnp.float32)

    out_ref[...] += partial.reshape(1, 1)


def kernel(embedded, centers):
    n, d = embedded.shape
    k, _ = centers.shape
    bn = 1024
    num_blocks = n // bn

    c2 = jnp.sum(centers * centers, axis=1)
    cta = jnp.concatenate(
        [centers.T, -0.5 * c2[None, :], jnp.full((1, k), -0.5, centers.dtype)],
        axis=0,
    ).astype(jnp.bfloat16)

    e2 = jnp.sum(embedded * embedded, axis=1, keepdims=True)
    ea = jnp.concatenate(
        [embedded, jnp.ones((n, 1), embedded.dtype), e2], axis=1
    ).astype(jnp.bfloat16)

    out = pl.pallas_call(
        functools.partial(_dcn_loss_kernel, n_total=float(n)),
        grid=(num_blocks,),
        in_specs=[
            pl.BlockSpec((bn, d + 2), lambda i: (i, 0)),
            pl.BlockSpec((d + 2, k), lambda i: (0, 0)),
        ],
        out_specs=pl.BlockSpec((1, 1), lambda i: (0, 0)),
        out_shape=jax.ShapeDtypeStruct((1, 1), jnp.float32),
    )(ea, cta)
    return out[0, 0]
